# baseline (speedup 1.0000x reference)
import functools

import jax
import jax.numpy as jnp
from jax import lax
from jax.experimental import pallas as pl
from jax.experimental.pallas import tpu as pltpu
from jax.experimental.pallas import tpu_sc as plsc

N_BATCH, C, D, H, W = 4, 128, 5, 75, 75
HW = H * W
HWP = 5632
PLANES = N_BATCH * D
RINTER = PLANES * HWP
NNZ = 40000
CHUNK = 256
NCH = 5
NC, NS, L = 2, 16, 16
NW = NC * NS
ZSLICE = (NC * RINTER) // NW
ZBUF = 704
HALF = CHUNK // 2


def _sc_body(feat, b_hbm, d_hbm, h_hbm, w_hbm, inter, valid,
             bbuf, dbuf, hbuf, wbuf, fbuf, destb, flagb, zbuf, obuf,
             zsem, isem, gsem, ssem):
    c = lax.axis_index("c")
    s = lax.axis_index("s")
    wid = s * NC + c

    zv = jnp.zeros((L,), jnp.float32)
    for k in range(ZBUF // L):
        zbuf[pl.ds(k * L, L)] = zv
    ov = jnp.full((L,), 1.0, jnp.float32)
    for k in range(HALF // L):
        obuf[pl.ds(k * L, L)] = ov

    zbase = c * RINTER + s * ZSLICE
    zcps = []
    for t in range(ZSLICE // ZBUF):
        cp = pltpu.make_async_copy(
            zbuf, valid.at[pl.ds(zbase + t * ZBUF, ZBUF)], zsem)
        cp.start()
        zcps.append(cp)

    vbs = [jnp.minimum((wid * NCH + j) * CHUNK, NNZ - CHUNK) for j in range(NCH)]
    icps = []
    for j in range(NCH):
        for src, dst in ((b_hbm, bbuf), (d_hbm, dbuf), (h_hbm, hbuf), (w_hbm, wbuf)):
            cp = pltpu.make_async_copy(
                src.at[pl.ds(vbs[j], CHUNK)], dst.at[pl.ds(j * CHUNK, CHUNK)], isem)
            cp.start()
            icps.append(cp)
    for cp in icps:
        cp.wait()

    for j in range(NCH):
        for v in range(CHUNK // L):
            off = v * L
            bb = bbuf[pl.ds(j * CHUNK + off, L)]
            dd = dbuf[pl.ds(j * CHUNK + off, L)]
            hh = hbuf[pl.ds(j * CHUNK + off, L)]
            ww = wbuf[pl.ds(j * CHUNK + off, L)]
            r = ((dd * H + hh) * W + ww) * N_BATCH + bb
            hv, lo = off // HALF, off % HALF
            destb[2 * j + hv, pl.ds(lo, L)] = r
            flagb[2 * j + hv, pl.ds(lo, L)] = r + c * RINTER

    for cp in zcps:
        cp.wait()
    plsc.subcore_barrier()

    def start_gather(j):
        cp = pltpu.make_async_copy(
            feat.at[pl.ds(vbs[j], CHUNK)],
            fbuf.at[pl.ds((j % 2) * CHUNK, CHUNK)], gsem)
        cp.start()
        return cp

    def start_scatters(j):
        cps = []
        for hv in range(2):
            cp = pltpu.make_async_copy(
                fbuf.at[pl.ds((j % 2) * CHUNK + hv * HALF, HALF)],
                inter.at[destb.at[2 * j + hv]], ssem)
            cp.start()
            cps.append(cp)
            cp = pltpu.make_async_copy(obuf, valid.at[flagb.at[2 * j + hv]], ssem)
            cp.start()
            cps.append(cp)
        return cps

    gcps, scps = [None] * NCH, [None] * NCH
    for j in range(NCH):
        if j >= 2:
            for cp in scps[j - 2]:
                cp.wait()
        gcps[j] = start_gather(j)
        if j >= 1:
            gcps[j - 1].wait()
            scps[j - 1] = start_scatters(j - 1)
    gcps[NCH - 1].wait()
    scps[NCH - 1] = start_scatters(NCH - 1)
    for j in (NCH - 2, NCH - 1):
        for cp in scps[j]:
            cp.wait()


@functools.partial(
    pl.kernel,
    out_type=(
        jax.ShapeDtypeStruct((RINTER, C), jnp.float32),
        jax.ShapeDtypeStruct((NC * RINTER,), jnp.float32),
    ),
    mesh=plsc.VectorSubcoreMesh(core_axis_name="c", subcore_axis_name="s"),
    scratch_types=[
        pltpu.VMEM((NCH * CHUNK,), jnp.int32),
        pltpu.VMEM((NCH * CHUNK,), jnp.int32),
        pltpu.VMEM((NCH * CHUNK,), jnp.int32),
        pltpu.VMEM((NCH * CHUNK,), jnp.int32),
        pltpu.VMEM((2 * CHUNK, C), jnp.float32),
        pltpu.VMEM((NCH * 2, HALF), jnp.int32),
        pltpu.VMEM((NCH * 2, HALF), jnp.int32),
        pltpu.VMEM((ZBUF,), jnp.float32),
        pltpu.VMEM((HALF,), jnp.float32),
        pltpu.SemaphoreType.DMA,
        pltpu.SemaphoreType.DMA,
        pltpu.SemaphoreType.DMA,
        pltpu.SemaphoreType.DMA,
    ],
)
def _sc_scatter(feat, b_hbm, d_hbm, h_hbm, w_hbm, inter, valid, *scratch):
    _sc_body(feat, b_hbm, d_hbm, h_hbm, w_hbm, inter, valid, *scratch)


KSEL = 1024
ROUT = D * HW * N_BATCH


def _sel_body(xref, vref, oref):
    x = xref[...]
    v = vref[0, :] + vref[1, :]
    oref[...] = jnp.where(v[:, None] > 0.0, x, 0.0)


def _tc_select(inter, valid2):
    return pl.pallas_call(
        _sel_body,
        grid=((ROUT + KSEL - 1) // KSEL,),
        in_specs=[
            pl.BlockSpec((KSEL, C), lambda g: (g, 0)),
            pl.BlockSpec((NC, KSEL), lambda g: (0, g)),
        ],
        out_specs=pl.BlockSpec((KSEL, C), lambda g: (g, 0)),
        out_shape=jax.ShapeDtypeStruct((ROUT, C), jnp.float32),
    )(inter, valid2)


def kernel(features, b_idx, d_idx, h_idx, w_idx):
    inter, valid = _sc_scatter(features, b_idx, d_idx, h_idx, w_idx)
    sel = _tc_select(inter, valid.reshape(NC, RINTER))
    x = sel.reshape(D, H, W, N_BATCH, C)
    return jnp.transpose(x, (3, 4, 0, 1, 2)).reshape(N_BATCH, C * D, H, W)


# speedup vs baseline: 1.1292x; 1.1292x over previous
import functools

import jax
import jax.numpy as jnp
from jax import lax
from jax.experimental import pallas as pl
from jax.experimental.pallas import tpu as pltpu
from jax.experimental.pallas import tpu_sc as plsc

N_BATCH, C, D, H, W = 4, 128, 5, 75, 75
HW = H * W
WP = 128
PLROWS = H * WP
PLANES = N_BATCH * D
RINTER = PLANES * PLROWS
NNZ = 40000
CHUNK = 256
NCH = 5
NC, NS, L = 2, 16, 16
NW = NC * NS
ZSLICE = (NC * RINTER) // NW
ZBUF = 1200
HALF = CHUNK // 2


def _sc_body(feat, b_hbm, d_hbm, h_hbm, w_hbm, inter, valid,
             bbuf, dbuf, hbuf, wbuf, fbuf, destb, flagb, zbuf, obuf,
             zsem, isem, gsem, ssem):
    c = lax.axis_index("c")
    s = lax.axis_index("s")
    wid = s * NC + c

    zv = jnp.zeros((L,), jnp.float32)
    for k in range(ZBUF // L):
        zbuf[pl.ds(k * L, L)] = zv
    ov = jnp.full((L,), 1.0, jnp.float32)
    for k in range(HALF // L):
        obuf[pl.ds(k * L, L)] = ov

    zbase = c * RINTER + s * ZSLICE
    zcps = []
    for t in range(ZSLICE // ZBUF):
        cp = pltpu.make_async_copy(
            zbuf, valid.at[pl.ds(zbase + t * ZBUF, ZBUF)], zsem)
        cp.start()
        zcps.append(cp)

    vbs = [jnp.minimum((wid * NCH + j) * CHUNK, NNZ - CHUNK) for j in range(NCH)]
    icps = []
    for j in range(NCH):
        for src, dst in ((b_hbm, bbuf), (d_hbm, dbuf), (h_hbm, hbuf), (w_hbm, wbuf)):
            cp = pltpu.make_async_copy(
                src.at[pl.ds(vbs[j], CHUNK)], dst.at[pl.ds(j * CHUNK, CHUNK)], isem)
            cp.start()
            icps.append(cp)
    for cp in icps:
        cp.wait()

    for j in range(NCH):
        for v in range(CHUNK // L):
            off = v * L
            bb = bbuf[pl.ds(j * CHUNK + off, L)]
            dd = dbuf[pl.ds(j * CHUNK + off, L)]
            hh = hbuf[pl.ds(j * CHUNK + off, L)]
            ww = wbuf[pl.ds(j * CHUNK + off, L)]
            r = ((bb * D + dd) * H + hh) * WP + ww
            hv, lo = off // HALF, off % HALF
            destb[2 * j + hv, pl.ds(lo, L)] = r
            flagb[2 * j + hv, pl.ds(lo, L)] = r + c * RINTER

    for cp in zcps:
        cp.wait()
    plsc.subcore_barrier()

    def start_gather(j):
        cp = pltpu.make_async_copy(
            feat.at[pl.ds(vbs[j], CHUNK)],
            fbuf.at[pl.ds((j % 2) * CHUNK, CHUNK)], gsem)
        cp.start()
        return cp

    def start_scatters(j):
        cps = []
        for hv in range(2):
            cp = pltpu.make_async_copy(
                fbuf.at[pl.ds((j % 2) * CHUNK + hv * HALF, HALF)],
                inter.at[destb.at[2 * j + hv]], ssem)
            cp.start()
            cps.append(cp)
            cp = pltpu.make_async_copy(obuf, valid.at[flagb.at[2 * j + hv]], ssem)
            cp.start()
            cps.append(cp)
        return cps

    gcps, scps = [None] * NCH, [None] * NCH
    for j in range(NCH):
        if j >= 2:
            for cp in scps[j - 2]:
                cp.wait()
        gcps[j] = start_gather(j)
        if j >= 1:
            gcps[j - 1].wait()
            scps[j - 1] = start_scatters(j - 1)
    gcps[NCH - 1].wait()
    scps[NCH - 1] = start_scatters(NCH - 1)
    for j in (NCH - 2, NCH - 1):
        for cp in scps[j]:
            cp.wait()


@functools.partial(
    pl.kernel,
    out_type=(
        jax.ShapeDtypeStruct((RINTER, C), jnp.float32),
        jax.ShapeDtypeStruct((NC * RINTER,), jnp.float32),
    ),
    mesh=plsc.VectorSubcoreMesh(core_axis_name="c", subcore_axis_name="s"),
    scratch_types=[
        pltpu.VMEM((NCH * CHUNK,), jnp.int32),
        pltpu.VMEM((NCH * CHUNK,), jnp.int32),
        pltpu.VMEM((NCH * CHUNK,), jnp.int32),
        pltpu.VMEM((NCH * CHUNK,), jnp.int32),
        pltpu.VMEM((2 * CHUNK, C), jnp.float32),
        pltpu.VMEM((NCH * 2, HALF), jnp.int32),
        pltpu.VMEM((NCH * 2, HALF), jnp.int32),
        pltpu.VMEM((ZBUF,), jnp.float32),
        pltpu.VMEM((HALF,), jnp.float32),
        pltpu.SemaphoreType.DMA,
        pltpu.SemaphoreType.DMA,
        pltpu.SemaphoreType.DMA,
        pltpu.SemaphoreType.DMA,
    ],
)
def _sc_scatter(feat, b_hbm, d_hbm, h_hbm, w_hbm, inter, valid, *scratch):
    _sc_body(feat, b_hbm, d_hbm, h_hbm, w_hbm, inter, valid, *scratch)


BC = 16
BCD = BC * D
NCW = C // BC


def _plane_copy(xref, raw, slot, plane, sem):
    return pltpu.make_async_copy(
        xref.at[pl.ds(plane * PLROWS, PLROWS)],
        raw.at[pl.ds(slot * PLROWS, PLROWS)], sem)


def _tc_body(xref, vref, oref, raw, tsp, sem):
    b = pl.program_id(0)
    cw = pl.program_id(1)

    @pl.when(cw == 0)
    def _stage():
        _plane_copy(xref, raw, 0, b * D, sem).start()
        for d in range(D):
            if d + 1 < D:
                _plane_copy(xref, raw, (d + 1) % 2, b * D + d + 1, sem).start()
            _plane_copy(xref, raw, d % 2, b * D + d, sem).wait()
            x = raw[pl.ds((d % 2) * PLROWS, PLROWS), :]
            v = (vref[0, pl.ds(d * PLROWS, PLROWS)]
                 + vref[1, pl.ds(d * PLROWS, PLROWS)])
            tsp[pl.ds(d * C, C), :] = jnp.where(v[None, :] > 0.0, x.T, 0.0)

    rows = jax.lax.broadcasted_iota(jnp.int32, (BCD, BC), 0)
    cols = jax.lax.broadcasted_iota(jnp.int32, (BCD, BC), 1)
    vblk = jnp.zeros((BCD, PLROWS), jnp.float32)
    for d in range(D):
        perm_d = jnp.logical_and(rows % D == d, rows // D == cols)
        x8 = tsp[pl.ds(d * C + BC * cw, BC), :]
        vblk = vblk + jax.lax.dot(perm_d.astype(jnp.float32), x8)
    for h in range(H):
        oref[0, :, h, :] = jax.lax.slice(vblk, (0, h * WP), (BCD, h * WP + W))


def _tc_format(inter, valid2):
    return pl.pallas_call(
        _tc_body,
        grid=(N_BATCH, NCW),
        in_specs=[
            pl.BlockSpec(memory_space=pl.ANY),
            pl.BlockSpec((NC, D * PLROWS), lambda b, cw: (0, b)),
        ],
        out_specs=pl.BlockSpec((1, BCD, H, W), lambda b, cw: (b, cw, 0, 0)),
        out_shape=jax.ShapeDtypeStruct((N_BATCH, C * D, H, W), jnp.float32),
        scratch_shapes=[
            pltpu.VMEM((2 * PLROWS, C), jnp.float32),
            pltpu.VMEM((D * C, PLROWS), jnp.float32),
            pltpu.SemaphoreType.DMA,
        ],
        compiler_params=pltpu.CompilerParams(vmem_limit_bytes=60 * 1024 * 1024),
    )(inter, valid2)


def kernel(features, b_idx, d_idx, h_idx, w_idx):
    inter, valid = _sc_scatter(features, b_idx, d_idx, h_idx, w_idx)
    return _tc_format(inter, valid.reshape(NC, RINTER))


# speedup vs baseline: 1.1353x; 1.0053x over previous
import functools

import jax
import jax.numpy as jnp
from jax import lax
from jax.experimental import pallas as pl
from jax.experimental.pallas import tpu as pltpu
from jax.experimental.pallas import tpu_sc as plsc

N_BATCH, C, D, H, W = 4, 128, 5, 75, 75
HW = H * W
WP = 128
PLROWS = H * WP
PLANES = N_BATCH * D
RINTER = PLANES * PLROWS
NNZ = 40000
CHUNK = 256
NCH = 5
NC, NS, L = 2, 16, 16
NW = NC * NS
ZSLICE = (NC * RINTER) // NW
ZBUF = 1200
HALF = CHUNK // 2


def _sc_body(feat, b_hbm, d_hbm, h_hbm, w_hbm, inter, valid,
             bbuf, dbuf, hbuf, wbuf, fbuf, destb, flagb, zbuf, obuf,
             zsem, isem, gsem, ssem):
    c = lax.axis_index("c")
    s = lax.axis_index("s")
    wid = s * NC + c

    zv = jnp.zeros((L,), jnp.float32)
    for k in range(ZBUF // L):
        zbuf[pl.ds(k * L, L)] = zv
    ov = jnp.full((L,), 1.0, jnp.float32)
    for k in range(HALF // L):
        obuf[pl.ds(k * L, L)] = ov

    zbase = c * RINTER + s * ZSLICE
    zcps = []
    for t in range(ZSLICE // ZBUF):
        cp = pltpu.make_async_copy(
            zbuf, valid.at[pl.ds(zbase + t * ZBUF, ZBUF)], zsem)
        cp.start()
        zcps.append(cp)

    vbs = [jnp.minimum((wid * NCH + j) * CHUNK, NNZ - CHUNK) for j in range(NCH)]
    icps = []
    for j in range(NCH):
        for src, dst in ((b_hbm, bbuf), (d_hbm, dbuf), (h_hbm, hbuf), (w_hbm, wbuf)):
            cp = pltpu.make_async_copy(
                src.at[pl.ds(vbs[j], CHUNK)], dst.at[pl.ds(j * CHUNK, CHUNK)], isem)
            cp.start()
            icps.append(cp)
    for cp in icps:
        cp.wait()

    for j in range(NCH):
        for v in range(CHUNK // L):
            off = v * L
            bb = bbuf[pl.ds(j * CHUNK + off, L)]
            dd = dbuf[pl.ds(j * CHUNK + off, L)]
            hh = hbuf[pl.ds(j * CHUNK + off, L)]
            ww = wbuf[pl.ds(j * CHUNK + off, L)]
            r = ((bb * D + dd) * H + hh) * WP + ww
            hv, lo = off // HALF, off % HALF
            destb[2 * j + hv, pl.ds(lo, L)] = r
            flagb[2 * j + hv, pl.ds(lo, L)] = r + c * RINTER

    for cp in zcps:
        cp.wait()
    plsc.subcore_barrier()

    def start_gather(j):
        cp = pltpu.make_async_copy(
            feat.at[pl.ds(vbs[j], CHUNK)],
            fbuf.at[pl.ds((j % 2) * CHUNK, CHUNK)], gsem)
        cp.start()
        return cp

    def start_scatters(j):
        cps = []
        for hv in range(2):
            cp = pltpu.make_async_copy(
                fbuf.at[pl.ds((j % 2) * CHUNK + hv * HALF, HALF)],
                inter.at[destb.at[2 * j + hv]], ssem)
            cp.start()
            cps.append(cp)
            cp = pltpu.make_async_copy(obuf, valid.at[flagb.at[2 * j + hv]], ssem)
            cp.start()
            cps.append(cp)
        return cps

    gcps, scps = [None] * NCH, [None] * NCH
    for j in range(NCH):
        if j >= 2:
            for cp in scps[j - 2]:
                cp.wait()
        gcps[j] = start_gather(j)
        if j >= 1:
            gcps[j - 1].wait()
            scps[j - 1] = start_scatters(j - 1)
    gcps[NCH - 1].wait()
    scps[NCH - 1] = start_scatters(NCH - 1)
    for j in (NCH - 2, NCH - 1):
        for cp in scps[j]:
            cp.wait()


@functools.partial(
    pl.kernel,
    out_type=(
        jax.ShapeDtypeStruct((RINTER, C), jnp.float32),
        jax.ShapeDtypeStruct((NC * RINTER,), jnp.float32),
    ),
    mesh=plsc.VectorSubcoreMesh(core_axis_name="c", subcore_axis_name="s"),
    scratch_types=[
        pltpu.VMEM((NCH * CHUNK,), jnp.int32),
        pltpu.VMEM((NCH * CHUNK,), jnp.int32),
        pltpu.VMEM((NCH * CHUNK,), jnp.int32),
        pltpu.VMEM((NCH * CHUNK,), jnp.int32),
        pltpu.VMEM((2 * CHUNK, C), jnp.float32),
        pltpu.VMEM((NCH * 2, HALF), jnp.int32),
        pltpu.VMEM((NCH * 2, HALF), jnp.int32),
        pltpu.VMEM((ZBUF,), jnp.float32),
        pltpu.VMEM((HALF,), jnp.float32),
        pltpu.SemaphoreType.DMA,
        pltpu.SemaphoreType.DMA,
        pltpu.SemaphoreType.DMA,
        pltpu.SemaphoreType.DMA,
    ],
)
def _sc_scatter(feat, b_hbm, d_hbm, h_hbm, w_hbm, inter, valid, *scratch):
    _sc_body(feat, b_hbm, d_hbm, h_hbm, w_hbm, inter, valid, *scratch)


BC = 16
BCD = BC * D
NCW = C // BC


def _tc_body(xref, vref, oref, tsp):
    s = pl.program_id(1)

    @pl.when(s < D)
    def _stage():
        for d in range(D):

            @pl.when(s == d)
            def _one():
                x = xref[...]
                v = vref[0, :] + vref[1, :]
                tsp[pl.ds(d * C, C), :] = jnp.where(v[None, :] > 0.0, x.T, 0.0)

    @pl.when(s >= D)
    def _emit():
        rows = jax.lax.broadcasted_iota(jnp.int32, (BCD, BC), 0)
        cols = jax.lax.broadcasted_iota(jnp.int32, (BCD, BC), 1)
        for w in range(NCW):

            @pl.when(s == D + w)
            def _win():
                vblk = jnp.zeros((BCD, PLROWS), jnp.float32)
                for d in range(D):
                    perm_d = jnp.logical_and(rows % D == d, rows // D == cols)
                    x8 = tsp[pl.ds(d * C + BC * w, BC), :]
                    vblk = vblk + jax.lax.dot(perm_d.astype(jnp.float32), x8)
                for h in range(H):
                    oref[0, :, h, :] = jax.lax.slice(
                        vblk, (0, h * WP), (BCD, h * WP + W))


def _tc_format(inter, valid2):
    nstep = D + NCW
    return pl.pallas_call(
        _tc_body,
        grid=(N_BATCH, nstep),
        in_specs=[
            pl.BlockSpec((PLROWS, C),
                         lambda b, s: (b * D + jnp.minimum(s, D - 1), 0)),
            pl.BlockSpec((NC, PLROWS),
                         lambda b, s: (0, b * D + jnp.minimum(s, D - 1))),
        ],
        out_specs=pl.BlockSpec(
            (1, BCD, H, W),
            lambda b, s: (b, jnp.maximum(s - D, 0), 0, 0)),
        out_shape=jax.ShapeDtypeStruct((N_BATCH, C * D, H, W), jnp.float32),
        scratch_shapes=[
            pltpu.VMEM((D * C, PLROWS), jnp.float32),
        ],
        compiler_params=pltpu.CompilerParams(vmem_limit_bytes=60 * 1024 * 1024),
    )(inter, valid2)


def kernel(features, b_idx, d_idx, h_idx, w_idx):
    inter, valid = _sc_scatter(features, b_idx, d_idx, h_idx, w_idx)
    return _tc_format(inter, valid.reshape(NC, RINTER))
